# baseline (speedup 1.0000x reference)
import functools

import jax
import jax.numpy as jnp
import numpy as np
from jax.experimental import pallas as pl
from jax.experimental.pallas import tpu as pltpu


@functools.lru_cache(maxsize=None)
def _w_sel(W):
    sel = np.zeros((W, W // 2), dtype=np.float32)
    sel[2 * np.arange(W // 2), np.arange(W // 2)] = 1.0
    return sel


def _make_body(Nb, C, H, W, pad):
    H_out, W_out = H // 2, W // 2

    def body(x_ref, sw_ref, o_ref):
        xe = x_ref[:, :, pl.ds(0, H_out, 2), :]
        y = jnp.dot(
            xe.reshape(Nb * C * H_out, W),
            sw_ref[...],
            preferred_element_type=jnp.float32,
        )
        o_ref[:, :pad] = jnp.zeros((Nb, pad, H_out, W_out), o_ref.dtype)
        o_ref[:, pad + C:] = jnp.zeros((Nb, pad, H_out, W_out), o_ref.dtype)
        o_ref[:, pad:pad + C] = y.reshape(Nb, C, H_out, W_out)

    return body


def _lambda_layer(x, planes):
    N, C, H, W = x.shape
    pad = planes // 4
    H_out, W_out = H // 2, W // 2
    C_out = C + 2 * pad

    sel = jnp.asarray(_w_sel(W))

    Nb = 32
    while N % Nb:
        Nb //= 2

    cost = pl.CostEstimate(
        flops=2 * (N * C * H_out) * W * W_out,
        transcendentals=0,
        bytes_accessed=4 * (x.size + sel.size + N * C_out * H_out * W_out),
    )

    out = pl.pallas_call(
        _make_body(Nb, C, H, W, pad),
        out_shape=jax.ShapeDtypeStruct((N, C_out, H_out, W_out), x.dtype),
        grid=(N // Nb,),
        in_specs=[
            pl.BlockSpec((Nb, C, H, W), lambda n: (n, 0, 0, 0)),
            pl.BlockSpec((W, W_out), lambda n: (0, 0)),
        ],
        out_specs=pl.BlockSpec((Nb, C_out, H_out, W_out), lambda n: (n, 0, 0, 0)),
        compiler_params=pltpu.CompilerParams(
            dimension_semantics=("parallel",),
            vmem_limit_bytes=48 << 20,
        ),
        cost_estimate=cost,
    )(x, sel)

    return out


def kernel(x):
    return _lambda_layer(x, planes=32)


# speedup vs baseline: 1.2461x; 1.2461x over previous
import jax
import jax.numpy as jnp
from jax.experimental import pallas as pl
from jax.experimental.pallas import tpu as pltpu


def _make_body(G, W_out, H_out, g_lo, g_hi):
    def body(x_ref, o_ref):
        g = pl.program_id(0)
        is_data = (g >= g_lo) & (g < g_hi)

        @pl.when(is_data)
        def _copy():
            o_ref[:, 0, :, :] = x_ref[:, 0, pl.ds(0, W_out, 2), :]

        @pl.when(jnp.logical_not(is_data))
        def _zero():
            o_ref[...] = jnp.zeros(o_ref.shape, o_ref.dtype)

    return body


def _lambda_layer(x, planes):
    N, C, H, W = x.shape
    pad = planes // 4
    H_out, W_out = H // 2, W // 2
    C_out = C + 2 * pad

    G = 4
    while pad % G or C % G:
        G //= 2
    g_lo, g_hi = pad // G, (pad + C) // G

    xt = jnp.transpose(x, (1, 2, 3, 0))

    Nb = 128 if N % 128 == 0 else N

    def x_map(g, h, n):
        is_data = (g >= g_lo) & (g < g_hi)
        gc = jnp.where(is_data, g - g_lo, 0)
        gh = jnp.where(is_data, 2 * h, 0)
        gn = jnp.where(is_data, n, 0)
        return (gc, gh, 0, gn)

    cost = pl.CostEstimate(
        flops=0,
        transcendentals=0,
        bytes_accessed=4 * (N * C * H_out * W + N * C_out * H_out * W_out),
    )

    out_t = pl.pallas_call(
        _make_body(G, W_out, H_out, g_lo, g_hi),
        out_shape=jax.ShapeDtypeStruct((C_out, H_out, W_out, N), x.dtype),
        grid=(C_out // G, H_out, N // Nb),
        in_specs=[pl.BlockSpec((G, 1, W, Nb), x_map)],
        out_specs=pl.BlockSpec((G, 1, W_out, Nb), lambda g, h, n: (g, h, 0, n)),
        compiler_params=pltpu.CompilerParams(
            dimension_semantics=("parallel", "parallel", "parallel"),
            vmem_limit_bytes=48 << 20,
        ),
        cost_estimate=cost,
    )(xt)

    return jnp.transpose(out_t, (3, 0, 1, 2))


def kernel(x):
    return _lambda_layer(x, planes=32)


# speedup vs baseline: 13.4419x; 10.7870x over previous
import functools

import jax
import jax.numpy as jnp
import numpy as np
from jax.experimental import pallas as pl
from jax.experimental.pallas import tpu as pltpu


@functools.lru_cache(maxsize=None)
def _w_sel(W):
    sel = np.zeros((W // 2, W), dtype=np.float32)
    sel[np.arange(W // 2), 2 * np.arange(W // 2)] = 1.0
    return sel


def _make_body(C, pad):
    def body(x_ref, sw_ref, o_ref):
        zpad = jnp.zeros(o_ref.shape[2:], o_ref.dtype)
        for c in range(pad):
            o_ref[c, 0] = zpad
            o_ref[pad + C + c, 0] = zpad
        for c in range(C):
            o_ref[pad + c, 0] = jnp.dot(
                sw_ref[...], x_ref[c, 0], preferred_element_type=jnp.float32
            )

    return body


def _lambda_layer(x, planes):
    N, C, H, W = x.shape
    pad = planes // 4
    H_out, W_out = H // 2, W // 2
    C_out = C + 2 * pad

    xt = jnp.transpose(x, (1, 2, 3, 0))
    sw = jnp.asarray(_w_sel(W))

    cost = pl.CostEstimate(
        flops=2 * C * H_out * W_out * W * N,
        transcendentals=0,
        bytes_accessed=4 * (N * C * H_out * W + N * C_out * H_out * W_out),
    )

    out_t = pl.pallas_call(
        _make_body(C, pad),
        out_shape=jax.ShapeDtypeStruct((C_out, H_out, W_out, N), x.dtype),
        grid=(H_out,),
        in_specs=[
            pl.BlockSpec((C, 1, W, N), lambda h: (0, 2 * h, 0, 0)),
            pl.BlockSpec((W_out, W), lambda h: (0, 0)),
        ],
        out_specs=pl.BlockSpec((C_out, 1, W_out, N), lambda h: (0, h, 0, 0)),
        compiler_params=pltpu.CompilerParams(
            dimension_semantics=("parallel",),
            vmem_limit_bytes=48 << 20,
        ),
        cost_estimate=cost,
    )(xt, sw)

    return jnp.transpose(out_t, (3, 0, 1, 2))


def kernel(x):
    return _lambda_layer(x, planes=32)
